# Initial kernel scaffold; baseline (speedup 1.0000x reference)
import jax
import jax.numpy as jnp
from jax.experimental import pallas as pl


def kernel(x, edge_index, batch, inp, gat_lin, att_src, att_dst, gat_bias, gcn_w, gcn_b, fc1_w, fc1_b, fc2_w, fc2_b, gc1_w, gc2_w, A):
    raise NotImplementedError("write your pallas kernel here")


# speedup vs baseline: 1.0000x; 1.0000x over previous
import jax
import jax.numpy as jnp
import numpy as np
from jax.experimental import pallas as pl

N = 10000
E = 160000
F = 78
H = 10
HF = 780
G = 512
C = 80
INC = 300


def _dense_tail_body(p_ref, w1_ref, b1_ref, w2_ref, b2_ref, yt_ref, o_ref):
    t = jnp.dot(p_ref[...], w1_ref[...], preferred_element_type=jnp.float32)
    t = jnp.maximum(t + b1_ref[...][None, :], 0.0)
    t = jnp.dot(t, w2_ref[...], preferred_element_type=jnp.float32)
    t = t + b2_ref[...][None, :]
    o_ref[...] = jnp.dot(t, yt_ref[...], preferred_element_type=jnp.float32)


def _dense_tail(p, w1, b1, w2, b2, y):
    return pl.pallas_call(
        _dense_tail_body,
        out_shape=jax.ShapeDtypeStruct((G, C), jnp.float32),
    )(p, w1, b1, w2, b2, y.T)


def _gen_adj(A):
    D = jnp.power(A.sum(1), -0.5)
    Dm = jnp.diag(D)
    return jnp.matmul(jnp.matmul(A, Dm).T, Dm)


def kernel(x, edge_index, batch, inp, gat_lin, att_src, att_dst, gat_bias,
           gcn_w, gcn_b, fc1_w, fc1_b, fc2_w, fc2_b, gc1_w, gc2_w, A):
    n = x.shape[0]
    loop = jnp.arange(n, dtype=edge_index.dtype)
    src = jnp.concatenate([edge_index[0], loop])
    dst = jnp.concatenate([edge_index[1], loop])
    h = jnp.matmul(x, gat_lin).reshape(n, H, F)
    a_src = (h * att_src[None, :, :]).sum(-1)
    a_dst = (h * att_dst[None, :, :]).sum(-1)
    e = jax.nn.leaky_relu(a_src[src] + a_dst[dst], negative_slope=0.2)
    m = jax.ops.segment_max(e, dst, num_segments=n)
    e = jnp.exp(e - m[dst])
    denom = jax.ops.segment_sum(e, dst, num_segments=n)
    alpha = e / (denom[dst] + 1e-16)
    msg = h[src] * alpha[:, :, None]
    x1 = jax.ops.segment_sum(msg, dst, num_segments=n).reshape(n, HF) + gat_bias
    x1 = jax.nn.relu(x1)
    ones = jnp.ones(src.shape, dtype=jnp.float32)
    deg = jax.ops.segment_sum(ones, dst, num_segments=n)
    dinv = jnp.where(deg > 0, jnp.power(deg, -0.5), 0.0)
    norm = dinv[src] * dinv[dst]
    s = jnp.matmul(x1, gcn_w)
    x2 = jax.ops.segment_sum(s[src] * norm[:, None], dst, num_segments=n) + gcn_b
    x2 = jax.nn.relu(x2)
    gm = jax.ops.segment_max(x2, batch, num_segments=G)
    gm = jnp.where(jnp.isfinite(gm), gm, 0.0)
    cnt = jax.ops.segment_sum(jnp.ones((n,), dtype=jnp.float32), batch, num_segments=G)
    ga = jax.ops.segment_sum(x2, batch, num_segments=G) / jnp.maximum(cnt, 1.0)[:, None]
    p = jnp.concatenate([gm, ga], axis=1)
    adj = jax.lax.stop_gradient(_gen_adj(A))
    y = jnp.matmul(adj, jnp.matmul(inp, gc1_w))
    y = jax.nn.leaky_relu(y, negative_slope=0.2)
    y = jnp.matmul(adj, jnp.matmul(y, gc2_w))
    return _dense_tail(p, fc1_w, fc1_b, fc2_w, fc2_b, y)
